# baseline (speedup 1.0000x reference)
import jax
import jax.numpy as jnp
from jax.experimental import pallas as pl
from jax.experimental.pallas import tpu as pltpu
from jax.sharding import Mesh, PartitionSpec as P


def _mm_kernel(m_ref, x_ref, o_ref, xb_ref):
    @pl.when(pl.program_id(0) == 0)
    def _():
        xb_ref[...] = x_ref[...].astype(jnp.bfloat16)

    o_ref[...] = jnp.dot(
        m_ref[...].astype(jnp.bfloat16),
        xb_ref[...],
        preferred_element_type=jnp.float32,
    )


def _matmul_shard(matrix, x):
    M, K = matrix.shape
    S = x.shape[1]
    bm = min(512, M)
    return pl.pallas_call(
        _mm_kernel,
        grid=(M // bm,),
        in_specs=[
            pl.BlockSpec((bm, K), lambda i: (i, 0)),
            pl.BlockSpec((K, S), lambda i: (0, 0)),
        ],
        out_specs=pl.BlockSpec((bm, S), lambda i: (i, 0)),
        out_shape=jax.ShapeDtypeStruct((M, S), jnp.float32),
        scratch_shapes=[pltpu.VMEM((K, S), jnp.bfloat16)],
        compiler_params=pltpu.CompilerParams(
            dimension_semantics=("arbitrary",),
        ),
    )(matrix, x)


def kernel(inp, matrix):
    B, C, S = inp.shape
    M, K = matrix.shape
    x = inp.reshape(B * C, S)

    devs = jax.devices()
    n = 1
    while n * 2 <= len(devs) and M % (n * 2) == 0:
        n *= 2

    if n == 1:
        out = _matmul_shard(matrix, x)
    else:
        mesh = Mesh(devs[:n], ("d",))
        out = jax.shard_map(
            _matmul_shard,
            mesh=mesh,
            in_specs=(P("d", None), P(None, None)),
            out_specs=P("d", None),
            check_vma=False,
        )(matrix, x)
    return out.reshape(B, C, S)


# speedup vs baseline: 8.3503x; 8.3503x over previous
import jax
import jax.numpy as jnp
from jax.experimental import pallas as pl
from jax.experimental.pallas import tpu as pltpu

_BM = 512


def _mm_kernel(m_ref, x_ref, o_ref, xb_ref):
    @pl.when(pl.program_id(0) == 0)
    def _():
        xb_ref[...] = x_ref[...].astype(jnp.bfloat16)

    o_ref[...] = jnp.dot(
        m_ref[...].astype(jnp.bfloat16),
        xb_ref[...],
        preferred_element_type=jnp.float32,
    )


def kernel(inp, matrix):
    B, C, S = inp.shape
    M, K = matrix.shape
    x = inp.reshape(B * C, S)
    out = pl.pallas_call(
        _mm_kernel,
        grid=(M // _BM,),
        in_specs=[
            pl.BlockSpec((_BM, K), lambda i: (i, 0)),
            pl.BlockSpec((K, S), lambda i: (0, 0)),
        ],
        out_specs=pl.BlockSpec((_BM, S), lambda i: (i, 0)),
        out_shape=jax.ShapeDtypeStruct((M, S), jnp.float32),
        scratch_shapes=[pltpu.VMEM((K, S), jnp.bfloat16)],
        compiler_params=pltpu.CompilerParams(
            dimension_semantics=("arbitrary",),
        ),
    )(matrix, x)
    return out.reshape(B, C, S)
